# baseline (device time: 6047 ns/iter reference)
import jax
import jax.numpy as jnp
from jax import lax
from jax.experimental import pallas as pl
from jax.experimental.pallas import tpu as pltpu

CHUNK_ROWS = (384, 128)


def kernel(x, dy, gamma):
    del gamma
    m, d = x.shape
    assert sum(CHUNK_ROWS) == m

    def body(
        x_hbm,
        dy_hbm,
        out_ref,
        xbuf,
        dybuf,
        xsems,
        ysems,
        comm_ref,
        send_sem,
        recv_sem,
        out_sem,
    ):
        my_x = lax.axis_index("x")
        my_y = lax.axis_index("y")
        my_z = lax.axis_index("z")
        partner = (1 - my_x, my_y, my_z)

        barrier_sem = pltpu.get_barrier_semaphore()
        pl.semaphore_signal(
            barrier_sem,
            inc=1,
            device_id=partner,
            device_id_type=pl.DeviceIdType.MESH,
        )

        offsets = [sum(CHUNK_ROWS[:c]) for c in range(len(CHUNK_ROWS))]
        copies = []
        for c, (off, rows) in enumerate(zip(offsets, CHUNK_ROWS)):
            sl = pl.ds(off, rows)
            cp_x = pltpu.make_async_copy(x_hbm.at[sl, :], xbuf.at[sl, :], xsems.at[c])
            cp_y = pltpu.make_async_copy(dy_hbm.at[sl, :], dybuf.at[sl, :], ysems.at[c])
            cp_x.start()
            cp_y.start()
            copies.append((cp_x, cp_y))

        dg = jnp.zeros((1, d), jnp.float32)
        db = jnp.zeros((1, d), jnp.float32)
        for c, (off, rows) in enumerate(zip(offsets, CHUNK_ROWS)):
            copies[c][0].wait()
            copies[c][1].wait()
            sl = pl.ds(off, rows)
            xv = xbuf[sl, :]
            dyv = dybuf[sl, :]
            mu = jnp.mean(xv, axis=1, keepdims=True)
            ex2 = jnp.mean(xv * xv, axis=1, keepdims=True)
            var = ex2 - mu * mu
            rstd = lax.rsqrt(var + 1e-5)
            dg = dg + jnp.sum(dyv * (xv * rstd - mu * rstd), axis=0, keepdims=True)
            db = db + jnp.sum(dyv, axis=0, keepdims=True)

        comm_ref[0, 0:1, :] = dg
        comm_ref[0, 1:2, :] = db

        pl.semaphore_wait(barrier_sem, 1)

        rdma = pltpu.make_async_remote_copy(
            src_ref=comm_ref.at[0],
            dst_ref=comm_ref.at[1],
            send_sem=send_sem,
            recv_sem=recv_sem,
            device_id=partner,
            device_id_type=pl.DeviceIdType.MESH,
        )
        rdma.start()
        rdma.wait()

        comm_ref[0, :, :] = comm_ref[0] + comm_ref[1]
        out_cp = pltpu.make_async_copy(comm_ref.at[0], out_ref, out_sem)
        out_cp.start()
        out_cp.wait()

    return pl.pallas_call(
        body,
        out_shape=jax.ShapeDtypeStruct((2, d), jnp.float32),
        in_specs=[
            pl.BlockSpec(memory_space=pl.ANY),
            pl.BlockSpec(memory_space=pl.ANY),
        ],
        out_specs=pl.BlockSpec(memory_space=pl.ANY),
        scratch_shapes=[
            pltpu.VMEM((m, d), jnp.float32),
            pltpu.VMEM((m, d), jnp.float32),
            pltpu.SemaphoreType.DMA((len(CHUNK_ROWS),)),
            pltpu.SemaphoreType.DMA((len(CHUNK_ROWS),)),
            pltpu.VMEM((2, 2, d), jnp.float32),
            pltpu.SemaphoreType.DMA,
            pltpu.SemaphoreType.DMA,
            pltpu.SemaphoreType.DMA,
        ],
        compiler_params=pltpu.CompilerParams(collective_id=0),
    )(
        pltpu.with_memory_space_constraint(x, pltpu.MemorySpace.HBM),
        pltpu.with_memory_space_constraint(dy, pltpu.MemorySpace.HBM),
    )


# device time: 5984 ns/iter; 1.0105x vs baseline; 1.0105x over previous
import jax
import jax.numpy as jnp
from jax import lax
from jax.experimental import pallas as pl
from jax.experimental.pallas import tpu as pltpu

CHUNK_ROWS = (256, 256)


def kernel(x, dy, gamma):
    del gamma
    m, d = x.shape
    assert sum(CHUNK_ROWS) == m

    def body(
        x_hbm,
        dy_hbm,
        out_ref,
        xbuf,
        dybuf,
        xsems,
        ysems,
        comm_ref,
        send_sem,
        recv_sem,
        out_sem,
    ):
        my_x = lax.axis_index("x")
        my_y = lax.axis_index("y")
        my_z = lax.axis_index("z")
        partner = (1 - my_x, my_y, my_z)

        barrier_sem = pltpu.get_barrier_semaphore()
        pl.semaphore_signal(
            barrier_sem,
            inc=1,
            device_id=partner,
            device_id_type=pl.DeviceIdType.MESH,
        )

        offsets = [sum(CHUNK_ROWS[:c]) for c in range(len(CHUNK_ROWS))]
        copies = []
        for c, (off, rows) in enumerate(zip(offsets, CHUNK_ROWS)):
            sl = pl.ds(off, rows)
            cp_x = pltpu.make_async_copy(x_hbm.at[sl, :], xbuf.at[sl, :], xsems.at[c])
            cp_y = pltpu.make_async_copy(dy_hbm.at[sl, :], dybuf.at[sl, :], ysems.at[c])
            cp_x.start()
            cp_y.start()
            copies.append((cp_x, cp_y))

        dg = jnp.zeros((1, d), jnp.float32)
        db = jnp.zeros((1, d), jnp.float32)
        for c, (off, rows) in enumerate(zip(offsets, CHUNK_ROWS)):
            copies[c][0].wait()
            copies[c][1].wait()
            sl = pl.ds(off, rows)
            xv = xbuf[sl, :]
            dyv = dybuf[sl, :]
            mu = jnp.mean(xv, axis=1, keepdims=True)
            ex2 = jnp.mean(xv * xv, axis=1, keepdims=True)
            var = ex2 - mu * mu
            rstd = lax.rsqrt(var + 1e-5)
            dg = dg + jnp.sum(dyv * (xv * rstd - mu * rstd), axis=0, keepdims=True)
            db = db + jnp.sum(dyv, axis=0, keepdims=True)

        comm_ref[0, 0:1, :] = dg
        comm_ref[0, 1:2, :] = db

        pl.semaphore_wait(barrier_sem, 1)

        rdma = pltpu.make_async_remote_copy(
            src_ref=comm_ref.at[0],
            dst_ref=comm_ref.at[1],
            send_sem=send_sem,
            recv_sem=recv_sem,
            device_id=partner,
            device_id_type=pl.DeviceIdType.MESH,
        )
        rdma.start()
        rdma.wait()

        comm_ref[0, :, :] = comm_ref[0] + comm_ref[1]
        out_cp = pltpu.make_async_copy(comm_ref.at[0], out_ref, out_sem)
        out_cp.start()
        out_cp.wait()

    return pl.pallas_call(
        body,
        out_shape=jax.ShapeDtypeStruct((2, d), jnp.float32),
        in_specs=[
            pl.BlockSpec(memory_space=pl.ANY),
            pl.BlockSpec(memory_space=pl.ANY),
        ],
        out_specs=pl.BlockSpec(memory_space=pl.ANY),
        scratch_shapes=[
            pltpu.VMEM((m, d), jnp.float32),
            pltpu.VMEM((m, d), jnp.float32),
            pltpu.SemaphoreType.DMA((len(CHUNK_ROWS),)),
            pltpu.SemaphoreType.DMA((len(CHUNK_ROWS),)),
            pltpu.VMEM((2, 2, d), jnp.float32),
            pltpu.SemaphoreType.DMA,
            pltpu.SemaphoreType.DMA,
            pltpu.SemaphoreType.DMA,
        ],
        compiler_params=pltpu.CompilerParams(collective_id=0),
    )(
        pltpu.with_memory_space_constraint(x, pltpu.MemorySpace.HBM),
        pltpu.with_memory_space_constraint(dy, pltpu.MemorySpace.HBM),
    )
